# baseline (speedup 1.0000x reference)
import functools

import jax
import jax.numpy as jnp
from jax import lax
from jax.experimental import pallas as pl
from jax.experimental.pallas import tpu as pltpu
from jax.experimental.pallas import tpu_sc as plsc

MAX_BATCH = 32
N_KV_HEADS = 8
Q_LEN = 16
HEAD_DIM = 128
_TOTAL = MAX_BATCH * N_KV_HEADS * Q_LEN * HEAD_DIM
_NC = 2
_NS = 16
_NW = _NC * _NS
_CHUNK = _TOTAL // _NW


@functools.partial(
    pl.kernel,
    mesh=plsc.VectorSubcoreMesh(core_axis_name="c", subcore_axis_name="s"),
    out_type=[
        jax.ShapeDtypeStruct((_TOTAL,), jnp.float32),
        jax.ShapeDtypeStruct((_TOTAL,), jnp.float32),
    ],
    scratch_types=[
        pltpu.SemaphoreType.DMA,
        pltpu.SemaphoreType.DMA,
    ],
)
def _sc_copy(k_hbm, v_hbm, ok_hbm, ov_hbm, sem_k, sem_v):
    wid = lax.axis_index("s") * _NC + lax.axis_index("c")
    sl = pl.ds(wid * _CHUNK, _CHUNK)
    ck = pltpu.async_copy(k_hbm.at[sl], ok_hbm.at[sl], sem_k)
    cv = pltpu.async_copy(v_hbm.at[sl], ov_hbm.at[sl], sem_v)
    ck.wait()
    cv.wait()


def kernel(k_new, v_new, k_cache, v_cache):
    del k_cache, v_cache
    ok, ov = _sc_copy(k_new.reshape(_TOTAL), v_new.reshape(_TOTAL))
    return (ok.reshape(k_new.shape), ov.reshape(v_new.shape))


# speedup vs baseline: 6.3155x; 6.3155x over previous
import functools

import jax
import jax.numpy as jnp
from jax import lax
from jax.experimental import pallas as pl
from jax.experimental.pallas import tpu as pltpu
from jax.experimental.pallas import tpu_sc as plsc

MAX_BATCH = 32
N_KV_HEADS = 8
Q_LEN = 16
HEAD_DIM = 128
_TOTAL = MAX_BATCH * N_KV_HEADS * Q_LEN * HEAD_DIM
_NC = 2
_NS = 16
_NW = _NC * _NS
_CHUNK = _TOTAL // _NW


@functools.partial(
    pl.kernel,
    mesh=plsc.VectorSubcoreMesh(core_axis_name="c", subcore_axis_name="s"),
    out_type=[
        jax.ShapeDtypeStruct((_TOTAL,), jnp.float32),
        jax.ShapeDtypeStruct((_TOTAL,), jnp.float32),
    ],
    scratch_types=[
        pltpu.VMEM((_CHUNK,), jnp.float32),
        pltpu.VMEM((_CHUNK,), jnp.float32),
        pltpu.SemaphoreType.DMA,
        pltpu.SemaphoreType.DMA,
    ],
)
def _sc_copy(k_hbm, v_hbm, ok_hbm, ov_hbm, kb, vb, sem_k, sem_v):
    wid = lax.axis_index("s") * _NC + lax.axis_index("c")
    sl = pl.ds(wid * _CHUNK, _CHUNK)
    ck = pltpu.async_copy(k_hbm.at[sl], kb, sem_k)
    cv = pltpu.async_copy(v_hbm.at[sl], vb, sem_v)
    ck.wait()
    cko = pltpu.async_copy(kb, ok_hbm.at[sl], sem_k)
    cv.wait()
    cvo = pltpu.async_copy(vb, ov_hbm.at[sl], sem_v)
    cko.wait()
    cvo.wait()


def kernel(k_new, v_new, k_cache, v_cache):
    del k_cache, v_cache
    ok, ov = _sc_copy(k_new.reshape(_TOTAL), v_new.reshape(_TOTAL))
    return (ok.reshape(k_new.shape), ov.reshape(v_new.shape))


# speedup vs baseline: 29.9486x; 4.7421x over previous
import jax
import jax.numpy as jnp
from jax.experimental import pallas as pl


def _copy_body(k_ref, v_ref, ok_ref, ov_ref):
    ok_ref[...] = k_ref[...]
    ov_ref[...] = v_ref[...]


def kernel(k_new, v_new, k_cache, v_cache):
    del k_cache, v_cache
    shape = jax.ShapeDtypeStruct(k_new.shape, k_new.dtype)
    out_k, out_v = pl.pallas_call(
        _copy_body,
        out_shape=[shape, shape],
    )(k_new, v_new)
    return (out_k, out_v)
